# Initial kernel scaffold; baseline (speedup 1.0000x reference)
import jax
import jax.numpy as jnp
from jax.experimental import pallas as pl


def kernel(x, edge_index, v_p, params):
    raise NotImplementedError("write your pallas kernel here")


# speedup vs baseline: 2.5987x; 2.5987x over previous
import functools

import jax
import jax.numpy as jnp
from jax import lax
from jax.experimental import pallas as pl
from jax.experimental.pallas import tpu as pltpu
from jax.experimental.pallas import tpu_sc as plsc

N = 10000
E = 320000
B = 16
EMB = 128
IN_F = 75
PH = 2560
LP = 256
HID = 1024
OUT = 256

NC = 2
NS = 16
NW = NC * NS
K = 128
CPW = 80
E_PAD = NW * CPW * K
NP = 10240
ROWS_T = NP // NS

_SC_MESH = plsc.VectorSubcoreMesh(
    core_axis_name="c", subcore_axis_name="s", num_cores=NC, num_subcores=NS)


def _sc_degree(src2, dst2, zeros_np):
  @functools.partial(
      pl.kernel,
      out_type=jax.ShapeDtypeStruct((NC, 2, NP, EMB), jnp.float32),
      mesh=_SC_MESH,
      scratch_types=[
          pltpu.VMEM((CPW, K), jnp.int32),
          pltpu.VMEM((CPW, K), jnp.int32),
          pltpu.VMEM((K, EMB), jnp.float32),
          pltpu.VMEM_SHARED((NP, EMB), jnp.float32),
      ])
  def deg_kernel(src_hbm, dst_hbm, z_hbm, out_hbm, src_v, dst_v, ones_v, acc):
    c = lax.axis_index("c")
    s = lax.axis_index("s")
    wid = c * NS + s
    row0 = s * ROWS_T
    ones16 = jnp.ones((16,), jnp.float32)

    def fill_body(i, carry):
      ones_v[i // 8, pl.ds((i % 8) * 16, 16)] = ones16
      return carry

    lax.fori_loop(0, K * 8, fill_body, 0)
    pltpu.sync_copy(src_hbm.at[pl.ds(wid * CPW, CPW)], src_v)
    pltpu.sync_copy(dst_hbm.at[pl.ds(wid * CPW, CPW)], dst_v)
    pltpu.sync_copy(z_hbm.at[pl.ds(row0, ROWS_T)], acc.at[pl.ds(row0, ROWS_T)])
    plsc.subcore_barrier()

    def body_o(j, carry):
      pltpu.sync_copy(ones_v, acc.at[src_v.at[j]], add=True)
      return carry

    lax.fori_loop(0, CPW, body_o, 0)
    plsc.subcore_barrier()
    pltpu.sync_copy(acc.at[pl.ds(row0, ROWS_T)],
                    out_hbm.at[c, 0, pl.ds(row0, ROWS_T)])
    pltpu.sync_copy(z_hbm.at[pl.ds(row0, ROWS_T)], acc.at[pl.ds(row0, ROWS_T)])
    plsc.subcore_barrier()

    def body_i(j, carry):
      pltpu.sync_copy(ones_v, acc.at[dst_v.at[j]], add=True)
      return carry

    lax.fori_loop(0, CPW, body_i, 0)
    plsc.subcore_barrier()
    pltpu.sync_copy(acc.at[pl.ds(row0, ROWS_T)],
                    out_hbm.at[c, 1, pl.ds(row0, ROWS_T)])

  return deg_kernel(src2, dst2, zeros_np)


def _sc_spmm(m_nodes, src2, dst2, zeros_np):
  @functools.partial(
      pl.kernel,
      out_type=jax.ShapeDtypeStruct((NC, NP, EMB), jnp.float32),
      mesh=_SC_MESH,
      scratch_types=[
          pltpu.VMEM((CPW, K), jnp.int32),
          pltpu.VMEM((CPW, K), jnp.int32),
          pltpu.VMEM((K, EMB), jnp.float32),
          pltpu.VMEM_SHARED((NP, EMB), jnp.float32),
          pltpu.SemaphoreType.DMA,
      ])
  def spmm_kernel(m_hbm, src_hbm, dst_hbm, z_hbm, out_hbm,
                  src_v, dst_v, rows_v, acc, sem):
    c = lax.axis_index("c")
    s = lax.axis_index("s")
    wid = c * NS + s
    row0 = s * ROWS_T
    pltpu.sync_copy(z_hbm.at[pl.ds(row0, ROWS_T)], acc.at[pl.ds(row0, ROWS_T)])
    pltpu.sync_copy(src_hbm.at[pl.ds(wid * CPW, CPW)], src_v)
    pltpu.sync_copy(dst_hbm.at[pl.ds(wid * CPW, CPW)], dst_v)
    plsc.subcore_barrier()

    def body(j, carry):
      pltpu.async_copy(m_hbm.at[src_v.at[j]], rows_v, sem).wait()
      pltpu.sync_copy(rows_v, acc.at[dst_v.at[j]], add=True)
      return carry

    lax.fori_loop(0, CPW, body, 0)
    plsc.subcore_barrier()
    pltpu.sync_copy(acc.at[pl.ds(row0, ROWS_T)],
                    out_hbm.at[c, pl.ds(row0, ROWS_T)])

  return spmm_kernel(m_nodes, src2, dst2, zeros_np)


_BR = 1024


def _tc_init(x_pad, w0_pad, degs4):
  def body(x_ref, w_ref, d_ref, h_ref, m_ref, no_ref, ni_ref):
    h = jnp.dot(x_ref[...], w_ref[...], preferred_element_type=jnp.float32)
    d = d_ref[...]
    deg_o = d[0] + d[2]
    deg_i = d[1] + d[3]
    no = lax.rsqrt(jnp.maximum(deg_o, 1.0))
    ni = lax.rsqrt(jnp.maximum(deg_i, 1.0))
    h_ref[...] = h
    m_ref[...] = h * no
    no_ref[...] = no
    ni_ref[...] = ni

  return pl.pallas_call(
      body,
      grid=(NP // _BR,),
      in_specs=[
          pl.BlockSpec((_BR, 80), lambda i: (i, 0)),
          pl.BlockSpec((80, EMB), lambda i: (0, 0)),
          pl.BlockSpec((4, _BR, EMB), lambda i: (0, i, 0)),
      ],
      out_specs=[pl.BlockSpec((_BR, EMB), lambda i: (i, 0))] * 4,
      out_shape=[jax.ShapeDtypeStruct((NP, EMB), jnp.float32)] * 4,
  )(x_pad, w0_pad, degs4)


def _tc_layer(aggp, h, ni_b, no_b, wg, bg, wr, br):
  def body(a_ref, h_ref, ni_ref, no_ref, wg_ref, bg_ref, wr_ref, br_ref,
           hn_ref, m_ref):
    a = a_ref[...]
    agg = (a[0] + a[1]) * ni_ref[...]
    hprev = h_ref[...]
    out = jnp.maximum(
        jnp.dot(agg, wg_ref[...], preferred_element_type=jnp.float32)
        + bg_ref[...], 0.0)
    res = jnp.maximum(
        jnp.dot(hprev, wr_ref[...], preferred_element_type=jnp.float32)
        + br_ref[...], 0.0)
    hn = out + res
    hn_ref[...] = hn
    m_ref[...] = hn * no_ref[...]

  return pl.pallas_call(
      body,
      grid=(NP // _BR,),
      in_specs=[
          pl.BlockSpec((NC, _BR, EMB), lambda i: (0, i, 0)),
          pl.BlockSpec((_BR, EMB), lambda i: (i, 0)),
          pl.BlockSpec((_BR, EMB), lambda i: (i, 0)),
          pl.BlockSpec((_BR, EMB), lambda i: (i, 0)),
          pl.BlockSpec((EMB, EMB), lambda i: (0, 0)),
          pl.BlockSpec((1, EMB), lambda i: (0, 0)),
          pl.BlockSpec((EMB, EMB), lambda i: (0, 0)),
          pl.BlockSpec((1, EMB), lambda i: (0, 0)),
      ],
      out_specs=[pl.BlockSpec((_BR, EMB), lambda i: (i, 0))] * 2,
      out_shape=[jax.ShapeDtypeStruct((NP, EMB), jnp.float32)] * 2,
  )(aggp, h, ni_b, no_b, wg, bg, wr, br)


def _tc_protein(vp_flat, wp, bp):
  BRP = 512
  steps = (B * LP) // BRP

  def body(x_ref, w_ref, b_ref, vp_ref, s_ref, q_ref):
    v = jnp.maximum(
        jnp.dot(x_ref[...], w_ref[...], preferred_element_type=jnp.float32)
        + b_ref[...], 0.0)
    vp_ref[...] = v.reshape(vp_ref.shape)
    ssum = jnp.broadcast_to(jnp.sum(v, axis=0, keepdims=True), (8, EMB))
    ssq = jnp.broadcast_to(jnp.sum(v * v, axis=0, keepdims=True), (8, EMB))
    i = pl.program_id(0)

    @pl.when(i == 0)
    def _():
      s_ref[...] = ssum
      q_ref[...] = ssq

    @pl.when(i != 0)
    def _():
      s_ref[...] += ssum
      q_ref[...] += ssq

  return pl.pallas_call(
      body,
      grid=(steps,),
      in_specs=[
          pl.BlockSpec((BRP, PH), lambda i: (i, 0)),
          pl.BlockSpec((PH, EMB), lambda i: (0, 0)),
          pl.BlockSpec((1, EMB), lambda i: (0, 0)),
      ],
      out_specs=[
          pl.BlockSpec((BRP // LP, LP, EMB), lambda i: (i, 0, 0)),
          pl.BlockSpec((8, EMB), lambda i: (0, 0)),
          pl.BlockSpec((8, EMB), lambda i: (0, 0)),
      ],
      out_shape=[
          jax.ShapeDtypeStruct((B, LP, EMB), jnp.float32),
          jax.ShapeDtypeStruct((8, EMB), jnp.float32),
          jax.ShapeDtypeStruct((8, EMB), jnp.float32),
      ],
  )(vp_flat, wp, bp)


def _tc_final(vp3, ssum, ssq, vd3, cls2, gp, bpn,
              w1, b1, g1, be1, w2, b2, g2, be2, w3, b3, g3, be3, w4, b4):
  n_tok = float(B * LP)
  inv_sqrt_d = 1.0 / (EMB ** 0.5)

  def body(vp_ref, s_ref, q_ref, vd_ref, cls_ref, gp_ref, bpn_ref,
           w1_ref, b1_ref, g1_ref, be1_ref, w2_ref, b2_ref, g2_ref, be2_ref,
           w3_ref, b3_ref, g3_ref, be3_ref, w4_ref, b4_ref, o_ref):
    def bf(v):
      return v.astype(jnp.bfloat16).astype(jnp.float32)

    mean = s_ref[0:1, :] / n_tok
    var = q_ref[0:1, :] / n_tok - mean * mean
    vp = ((vp_ref[...] - mean[None]) / jnp.sqrt(var + 1e-5)[None]
          * gp_ref[...][None] + bpn_ref[...][None])
    vd = vd_ref[...]
    vp_b = bf(vp)
    vd_b = bf(vd)
    cls_b = bf(cls_ref[...])[None]
    sqrt_d = jnp.sqrt(jnp.float32(EMB))
    sp = jnp.sum(vp_b * cls_b, axis=2) / sqrt_d
    sd = jnp.sum(vd_b * cls_b, axis=2) / sqrt_d
    mx = jnp.maximum(jnp.max(sp, axis=1, keepdims=True),
                     jnp.max(sd, axis=1, keepdims=True))
    ep = jnp.exp(sp - mx)
    ed = jnp.exp(sd - mx)
    den = (jnp.sum(ep, axis=1, keepdims=True)
           + jnp.sum(ed, axis=1, keepdims=True))
    ap = bf(ep / den)
    ad = bf(ed / den)
    z = (jnp.sum(ap[:, :, None] * vp_b, axis=1)
         + jnp.sum(ad[:, :, None] * vd_b, axis=1))

    def bn(hh, g, bb):
      mu = jnp.mean(hh, axis=0, keepdims=True)
      va = jnp.mean((hh - mu) * (hh - mu), axis=0, keepdims=True)
      return (hh - mu) / jnp.sqrt(va + 1e-5) * g + bb

    h1 = bn(jnp.maximum(
        jnp.dot(z, w1_ref[...], preferred_element_type=jnp.float32)
        + b1_ref[...], 0.0), g1_ref[...], be1_ref[...])
    h2 = bn(jnp.maximum(
        jnp.dot(h1, w2_ref[...], preferred_element_type=jnp.float32)
        + b2_ref[...], 0.0), g2_ref[...], be2_ref[...])
    h3 = bn(jnp.maximum(
        jnp.dot(h2, w3_ref[...], preferred_element_type=jnp.float32)
        + b3_ref[...], 0.0), g3_ref[...], be3_ref[...])
    o_ref[...] = (jnp.dot(h3, w4_ref[...], preferred_element_type=jnp.float32)
                  + b4_ref[...])

  nd = N // B
  full = lambda shp: pl.BlockSpec(shp, lambda: tuple(0 for _ in shp))
  return pl.pallas_call(
      body,
      grid=(),
      in_specs=[
          full((B, LP, EMB)), full((8, EMB)), full((8, EMB)),
          full((B, nd, EMB)), full((1, EMB)), full((1, EMB)), full((1, EMB)),
          full((EMB, HID)), full((1, HID)), full((1, HID)), full((1, HID)),
          full((HID, HID)), full((1, HID)), full((1, HID)), full((1, HID)),
          full((HID, OUT)), full((1, OUT)), full((1, OUT)), full((1, OUT)),
          full((OUT, 1)), full((1, 1)),
      ],
      out_specs=full((B, 1)),
      out_shape=jax.ShapeDtypeStruct((B, 1), jnp.float32),
  )(vp3, ssum, ssq, vd3, cls2, gp, bpn,
    w1, b1, g1, be1, w2, b2, g2, be2, w3, b3, g3, be3, w4, b4)


def kernel(x, edge_index, v_p, params):
  src = edge_index[0]
  dst = edge_index[1]
  pad = E_PAD - E
  src2 = jnp.concatenate([src, jnp.full((pad,), N, jnp.int32)]).reshape(-1, K)
  dst2 = jnp.concatenate([dst, jnp.full((pad,), N, jnp.int32)]).reshape(-1, K)
  x_pad = jnp.pad(x, ((0, NP - N), (0, 80 - IN_F)))
  w0_pad = jnp.pad(params['W0'], ((0, 80 - IN_F), (0, 0)))
  zeros_np = jnp.zeros((NP, EMB), jnp.float32)

  degs = _sc_degree(src2, dst2, zeros_np)
  degs4 = degs.reshape(4, NP, EMB)
  h, m, no_b, ni_b = _tc_init(x_pad, w0_pad, degs4)
  for layer in params['gcn']:
    aggp = _sc_spmm(m, src2, dst2, zeros_np)
    h, m = _tc_layer(aggp, h, ni_b, no_b,
                     layer['Wg'], layer['bg'].reshape(1, -1),
                     layer['Wr'], layer['br'].reshape(1, -1))

  vp3, ssum, ssq = _tc_protein(v_p.reshape(B * LP, PH), params['Wp'],
                               params['bp'].reshape(1, -1))
  vd3 = h[:N].reshape(B, N // B, EMB)
  out = _tc_final(
      vp3, ssum, ssq, vd3, params['cls'].reshape(1, EMB),
      params['gp'].reshape(1, -1), params['bpn'].reshape(1, -1),
      params['W1'], params['b1'].reshape(1, -1),
      params['g1'].reshape(1, -1), params['be1'].reshape(1, -1),
      params['W2'], params['b2'].reshape(1, -1),
      params['g2'].reshape(1, -1), params['be2'].reshape(1, -1),
      params['W3'], params['b3'].reshape(1, -1),
      params['g3'].reshape(1, -1), params['be3'].reshape(1, -1),
      params['W4'], params['b4'].reshape(1, -1))
  return out
